# Initial kernel scaffold; baseline (speedup 1.0000x reference)
import jax
import jax.numpy as jnp
from jax.experimental import pallas as pl


def kernel(x, edge_index, W_l1, W_r1, b1, W_l2, W_r2, b2):
    raise NotImplementedError("write your pallas kernel here")


# speedup vs baseline: 5.4355x; 5.4355x over previous
import functools

import jax
import jax.numpy as jnp
from jax import lax
from jax.experimental import pallas as pl
from jax.experimental.pallas import tpu as pltpu
from jax.experimental.pallas import tpu_sc as plsc

_N = 10000
_E = 320000
_D = 128
_NC = 2
_NS = 16
_NW = _NC * _NS
_EPW = _E // _NW
_CHUNK = 80
_NCHUNK = _EPW // _CHUNK
_RPT = 1000
_ZROWS = 200
_CNTC = 1000


def _zeros16():
    return jnp.zeros((16,), jnp.float32)


def _make_sc_agg(with_cnt: bool):
    mesh = plsc.VectorSubcoreMesh(core_axis_name="c", subcore_axis_name="s",
                                  num_cores=_NC, num_subcores=_NS)
    out_type = [jax.ShapeDtypeStruct((_NC, _N, _D), jnp.float32)]
    if with_cnt:
        out_type.append(jax.ShapeDtypeStruct((_NC * _N,), jnp.float32))
    scratch = [
        pltpu.VMEM((_CHUNK,), jnp.int32),
        pltpu.VMEM((_CHUNK,), jnp.int32),
        pltpu.VMEM((_CHUNK, _D), jnp.float32),
        pltpu.VMEM((_ZROWS, _D), jnp.float32),
        pltpu.VMEM((_CHUNK,), jnp.float32),
        pltpu.VMEM((_CNTC,), jnp.float32),
        pltpu.VMEM_SHARED((_N, _D), jnp.float32),
        pltpu.VMEM_SHARED((_N,), jnp.float32),
        pltpu.SemaphoreType.DMA,
    ]

    def body(src_hbm, dst_hbm, x_hbm, *refs):
        if with_cnt:
            agg_out, cnt_out = refs[0], refs[1]
            rest = refs[2:]
        else:
            agg_out = refs[0]
            rest = refs[1:]
        (src_v, dst_v, rows_v, zrow_v, ones_v, zcnt_v,
         agg_sh, cnt_sh, sem) = rest

        cid = lax.axis_index("c")
        sid = lax.axis_index("s")
        wid = sid * _NC + cid

        z16 = _zeros16()

        def fill_zrow(i, _):
            r = i // 8
            col = (i % 8) * 16
            zrow_v[r, pl.ds(col, 16)] = z16
            return 0

        lax.fori_loop(0, _ZROWS * 8, fill_zrow, 0)

        if with_cnt:
            o16 = jnp.ones((16,), jnp.float32)

            def fill_ones(i, _):
                ones_v[pl.ds(i * 16, 16)] = o16
                return 0

            lax.fori_loop(0, _CHUNK // 16, fill_ones, 0)

            def fill_zcnt(i, _):
                zcnt_v[pl.ds(i * 16, 16)] = z16
                return 0

            lax.fori_loop(0, _CNTC // 16, fill_zcnt, 0)

        @pl.when(sid < _N // _RPT)
        def _():
            for k in range(_RPT // _ZROWS):
                pltpu.sync_copy(
                    zrow_v, agg_sh.at[pl.ds(sid * _RPT + k * _ZROWS, _ZROWS)])
            if with_cnt:
                pltpu.sync_copy(zcnt_v, cnt_sh.at[pl.ds(sid * _CNTC, _CNTC)])
        plsc.subcore_barrier()

        base0 = wid * _EPW

        def step(c, _):
            base = pl.multiple_of(base0 + c * _CHUNK, 8)
            pltpu.sync_copy(src_hbm.at[pl.ds(base, _CHUNK)], src_v)
            pltpu.sync_copy(dst_hbm.at[pl.ds(base, _CHUNK)], dst_v)
            pltpu.async_copy(x_hbm.at[src_v], rows_v, sem).wait()
            pltpu.sync_copy(rows_v, agg_sh.at[dst_v], add=True)
            if with_cnt:
                pltpu.sync_copy(ones_v, cnt_sh.at[dst_v], add=True)
            return 0

        lax.fori_loop(0, _NCHUNK, step, 0)

        plsc.subcore_barrier()

        @pl.when(sid < _N // _RPT)
        def _():
            for k in range(_RPT // _ZROWS):
                rs = sid * _RPT + k * _ZROWS
                pltpu.sync_copy(agg_sh.at[pl.ds(rs, _ZROWS)],
                                agg_out.at[cid, pl.ds(rs, _ZROWS)])
            if with_cnt:
                pltpu.sync_copy(cnt_sh.at[pl.ds(sid * _CNTC, _CNTC)], zcnt_v)
                pltpu.sync_copy(
                    zcnt_v, cnt_out.at[pl.ds(cid * _N + sid * _CNTC, _CNTC)])

    return pl.kernel(body, out_type=out_type, mesh=mesh, scratch_types=scratch,
                     name="sc_sage_agg_cnt" if with_cnt else "sc_sage_agg")


_make_sc_agg = functools.lru_cache(maxsize=None)(_make_sc_agg)


def _make_tc_dense(relu: bool):
    bm = 1000
    grid = (_N // bm,)

    def body(agg_ref, cnt_ref, x_ref, wl_ref, wr_ref, b_ref, o_ref):
        a = agg_ref[0] + agg_ref[1]
        c = cnt_ref[0] + cnt_ref[1]
        mean = a / jnp.maximum(c, 1.0)
        y = (jnp.dot(mean, wl_ref[...], preferred_element_type=jnp.float32)
             + jnp.dot(x_ref[...], wr_ref[...], preferred_element_type=jnp.float32)
             + b_ref[...])
        if relu:
            y = jnp.maximum(y, 0.0)
        o_ref[...] = y

    return pl.pallas_call(
        body,
        grid=grid,
        in_specs=[
            pl.BlockSpec((_NC, bm, _D), lambda i: (0, i, 0)),
            pl.BlockSpec((_NC, bm, 1), lambda i: (0, i, 0)),
            pl.BlockSpec((bm, _D), lambda i: (i, 0)),
            pl.BlockSpec((_D, _D), lambda i: (0, 0)),
            pl.BlockSpec((_D, _D), lambda i: (0, 0)),
            pl.BlockSpec((1, _D), lambda i: (0, 0)),
        ],
        out_specs=pl.BlockSpec((bm, _D), lambda i: (i, 0)),
        out_shape=jax.ShapeDtypeStruct((_N, _D), jnp.float32),
        name="tc_sage_dense_relu" if relu else "tc_sage_dense",
    )


_tc_dense_relu = _make_tc_dense(True)
_tc_dense = _make_tc_dense(False)


@jax.jit
def kernel(x, edge_index, W_l1, W_r1, b1, W_l2, W_r2, b2):
    src = edge_index[0].astype(jnp.int32)
    dst = edge_index[1].astype(jnp.int32)

    agg1, cnt = _make_sc_agg(True)(src, dst, x)
    cnt3 = cnt.reshape(_NC, _N, 1)
    h = _tc_dense_relu(agg1, cnt3, x, W_l1, W_r1, b1.reshape(1, _D))
    (agg2,) = _make_sc_agg(False)(src, dst, h)
    out = _tc_dense(agg2, cnt3, h, W_l2, W_r2, b2.reshape(1, _D))
    return out


# speedup vs baseline: 12.3803x; 2.2777x over previous
import functools

import jax
import jax.numpy as jnp
from jax import lax
from jax.experimental import pallas as pl
from jax.experimental.pallas import tpu as pltpu
from jax.experimental.pallas import tpu_sc as plsc

_N = 10000
_E = 320000
_D = 128
_NC = 2
_NS = 16
_NW = _NC * _NS
_EPW = _E // _NW
_CHUNK = 80
_NCHUNK = _EPW // _CHUNK
_RPT = 1000
_ZROWS = 200
_CNTC = 1000
_NBUF = 2


def _zeros16():
    return jnp.zeros((16,), jnp.float32)


def _make_sc_agg(with_cnt: bool):
    mesh = plsc.VectorSubcoreMesh(core_axis_name="c", subcore_axis_name="s",
                                  num_cores=_NC, num_subcores=_NS)
    out_type = [jax.ShapeDtypeStruct((_NC, _N, _D), jnp.float32)]
    if with_cnt:
        out_type.append(jax.ShapeDtypeStruct((_NC * _N,), jnp.float32))
    scratch = (
        [pltpu.VMEM((_EPW,), jnp.int32),
         pltpu.VMEM((_NCHUNK, _CHUNK), jnp.int32)]
        + [pltpu.VMEM((_CHUNK, _D), jnp.float32)] * _NBUF
        + [pltpu.VMEM((_CHUNK,), jnp.float32),
           pltpu.VMEM((_CNTC,), jnp.float32),
           pltpu.VMEM_SHARED((_N, _D), jnp.float32),
           pltpu.VMEM_SHARED((_N,), jnp.float32)]
        + [pltpu.SemaphoreType.DMA] * (2 * _NBUF)
        + [pltpu.SemaphoreType.DMA] * _NBUF
    )

    def body(src_hbm, dst_hbm, x_hbm, *refs):
        if with_cnt:
            agg_out, cnt_out = refs[0], refs[1]
            rest = refs[2:]
        else:
            agg_out = refs[0]
            rest = refs[1:]
        src_v, dst_v = rest[0], rest[1]
        rows = rest[2:2 + _NBUF]
        ones_v, zcnt_v, agg_sh, cnt_sh = rest[2 + _NBUF:6 + _NBUF]
        gsem = rest[6 + _NBUF:6 + 2 * _NBUF]
        ssem = rest[6 + 2 * _NBUF:6 + 3 * _NBUF]
        csem = rest[6 + 3 * _NBUF:6 + 4 * _NBUF]

        cid = lax.axis_index("c")
        sid = lax.axis_index("s")
        wid = sid * _NC + cid

        sbase = pl.multiple_of(wid * _EPW, 8)
        idx_cp = [pltpu.async_copy(src_hbm.at[pl.ds(sbase, _EPW)], src_v,
                                   gsem[0]),
                  pltpu.async_copy(dst_hbm.at[wid], dst_v, gsem[1])]

        z16 = _zeros16()

        def fill_zrow(i, _):
            r = i // 8
            col = (i % 8) * 16
            rows[0][r, pl.ds(col, 16)] = z16
            return 0

        lax.fori_loop(0, _CHUNK * 8, fill_zrow, 0)

        if with_cnt:
            o16 = jnp.ones((16,), jnp.float32)

            def fill_ones(i, _):
                ones_v[pl.ds(i * 16, 16)] = o16
                return 0

            lax.fori_loop(0, _CHUNK // 16, fill_ones, 0)

            def fill_zcnt(i, _):
                zcnt_v[pl.ds(i * 16, 16)] = z16
                return 0

            lax.fori_loop(0, _CNTC // 16, fill_zcnt, 0)

        @pl.when(sid < _N // _RPT)
        def _():
            for k in range(_RPT // _CHUNK):
                pltpu.sync_copy(
                    rows[0], agg_sh.at[pl.ds(sid * _RPT + k * _CHUNK, _CHUNK)])
            pltpu.sync_copy(
                rows[0].at[pl.ds(0, _RPT - (_RPT // _CHUNK) * _CHUNK)],
                agg_sh.at[pl.ds(sid * _RPT + (_RPT // _CHUNK) * _CHUNK,
                                _RPT - (_RPT // _CHUNK) * _CHUNK)])
            if with_cnt:
                pltpu.sync_copy(zcnt_v, cnt_sh.at[pl.ds(sid * _CNTC, _CNTC)])
        for cp in idx_cp:
            cp.wait()
        plsc.subcore_barrier()

        def start_gather(c, b):
            off = pl.multiple_of(c * _CHUNK, 8)
            pltpu.async_copy(x_hbm.at[src_v.at[pl.ds(off, _CHUNK)]], rows[b],
                             gsem[b])

        def wait_gather(b):
            pltpu.make_async_copy(x_hbm.at[src_v.at[pl.ds(0, _CHUNK)]],
                                  rows[b], gsem[b]).wait()

        def start_scatter(c, b):
            pltpu.async_copy(rows[b], agg_sh.at[dst_v.at[c]], ssem[b],
                             add=True)
            if with_cnt:
                pltpu.async_copy(ones_v, cnt_sh.at[dst_v.at[c]], csem[b],
                                 add=True)

        def wait_scatter(b):
            pltpu.make_async_copy(rows[b], agg_sh.at[dst_v.at[0]],
                                  ssem[b]).wait()
            if with_cnt:
                pltpu.make_async_copy(ones_v, cnt_sh.at[dst_v.at[0]],
                                      csem[b]).wait()

        for b in range(_NBUF):
            start_gather(b, b)
            if b >= 1:
                wait_gather(b - 1)
                start_scatter(b - 1, b - 1)

        def group(g, _):
            for b in range(_NBUF):
                c = g * _NBUF + b
                wait_scatter(b)
                start_gather(c, b)
                bprev = (b - 1) % _NBUF
                wait_gather(bprev)
                start_scatter(c - 1, bprev)
            return 0

        lax.fori_loop(1, _NCHUNK // _NBUF, group, 0)

        for c in range((_NCHUNK // _NBUF) * _NBUF, _NCHUNK):
            b = c % _NBUF
            wait_scatter(b)
            start_gather(c, b)
            wait_gather((b - 1) % _NBUF)
            start_scatter(c - 1, (b - 1) % _NBUF)
        blast = (_NCHUNK - 1) % _NBUF
        wait_gather(blast)
        start_scatter(_NCHUNK - 1, blast)
        for b in range(_NBUF):
            wait_scatter(b)

        plsc.subcore_barrier()

        @pl.when(sid < _N // _RPT)
        def _():
            for k in range(_RPT // _ZROWS):
                rs = sid * _RPT + k * _ZROWS
                pltpu.sync_copy(agg_sh.at[pl.ds(rs, _ZROWS)],
                                agg_out.at[cid, pl.ds(rs, _ZROWS)])
            if with_cnt:
                pltpu.sync_copy(cnt_sh.at[pl.ds(sid * _CNTC, _CNTC)], zcnt_v)
                pltpu.sync_copy(
                    zcnt_v, cnt_out.at[pl.ds(cid * _N + sid * _CNTC, _CNTC)])

    return pl.kernel(body, out_type=out_type, mesh=mesh, scratch_types=scratch,
                     name="sc_sage_agg_cnt" if with_cnt else "sc_sage_agg")


_make_sc_agg = functools.lru_cache(maxsize=None)(_make_sc_agg)


def _make_tc_dense(relu: bool):
    bm = 1000
    grid = (_N // bm,)

    def body(agg_ref, cnt_ref, x_ref, wl_ref, wr_ref, b_ref, o_ref):
        a = agg_ref[0] + agg_ref[1]
        c = cnt_ref[0] + cnt_ref[1]
        mean = a / jnp.maximum(c, 1.0)
        y = (jnp.dot(mean, wl_ref[...], preferred_element_type=jnp.float32)
             + jnp.dot(x_ref[...], wr_ref[...], preferred_element_type=jnp.float32)
             + b_ref[...])
        if relu:
            y = jnp.maximum(y, 0.0)
        o_ref[...] = y

    return pl.pallas_call(
        body,
        grid=grid,
        in_specs=[
            pl.BlockSpec((_NC, bm, _D), lambda i: (0, i, 0)),
            pl.BlockSpec((_NC, bm, 1), lambda i: (0, i, 0)),
            pl.BlockSpec((bm, _D), lambda i: (i, 0)),
            pl.BlockSpec((_D, _D), lambda i: (0, 0)),
            pl.BlockSpec((_D, _D), lambda i: (0, 0)),
            pl.BlockSpec((1, _D), lambda i: (0, 0)),
        ],
        out_specs=pl.BlockSpec((bm, _D), lambda i: (i, 0)),
        out_shape=jax.ShapeDtypeStruct((_N, _D), jnp.float32),
        name="tc_sage_dense_relu" if relu else "tc_sage_dense",
    )


_tc_dense_relu = _make_tc_dense(True)
_tc_dense = _make_tc_dense(False)


@jax.jit
def kernel(x, edge_index, W_l1, W_r1, b1, W_l2, W_r2, b2):
    src = edge_index[0].astype(jnp.int32)
    dst = edge_index[1].astype(jnp.int32).reshape(_NW, _NCHUNK, _CHUNK)

    agg1, cnt = _make_sc_agg(True)(src, dst, x)
    cnt3 = cnt.reshape(_NC, _N, 1)
    h = _tc_dense_relu(agg1, cnt3, x, W_l1, W_r1, b1.reshape(1, _D))
    (agg2,) = _make_sc_agg(False)(src, dst, h)
    out = _tc_dense(agg2, cnt3, h, W_l2, W_r2, b2.reshape(1, _D))
    return out


# speedup vs baseline: 12.4481x; 1.0055x over previous
import functools

import jax
import jax.numpy as jnp
from jax import lax
from jax.experimental import pallas as pl
from jax.experimental.pallas import tpu as pltpu
from jax.experimental.pallas import tpu_sc as plsc

_N = 10000
_E = 320000
_D = 128
_NC = 2
_NS = 16
_NW = _NC * _NS
_EPW = _E // _NW
_CHUNK = 80
_NCHUNK = _EPW // _CHUNK
_RPT = 1000
_ZROWS = 200
_CNTC = 1000
_NBUF = 2


def _zeros16():
    return jnp.zeros((16,), jnp.float32)


def _make_sc_agg(with_cnt: bool):
    mesh = plsc.VectorSubcoreMesh(core_axis_name="c", subcore_axis_name="s",
                                  num_cores=_NC, num_subcores=_NS)
    out_type = [jax.ShapeDtypeStruct((_NC, _N, _D), jnp.float32)]
    if with_cnt:
        out_type.append(jax.ShapeDtypeStruct((_NC * _N,), jnp.float32))
    scratch = (
        [pltpu.VMEM((_EPW,), jnp.int32),
         pltpu.VMEM((_NCHUNK, _CHUNK), jnp.int32)]
        + [pltpu.VMEM((_CHUNK, _D), jnp.float32)] * _NBUF
        + [pltpu.VMEM((_CHUNK,), jnp.float32),
           pltpu.VMEM((_CNTC,), jnp.float32),
           pltpu.VMEM_SHARED((_N, _D), jnp.float32),
           pltpu.VMEM_SHARED((_N,), jnp.float32)]
        + [pltpu.SemaphoreType.DMA] * (2 * _NBUF)
        + [pltpu.SemaphoreType.DMA] * _NBUF
    )

    def body(src_hbm, dst_hbm, x_hbm, *refs):
        if with_cnt:
            agg_out, cnt_out = refs[0], refs[1]
            rest = refs[2:]
        else:
            agg_out = refs[0]
            rest = refs[1:]
        src_v, dst_v = rest[0], rest[1]
        rows = rest[2:2 + _NBUF]
        ones_v, zcnt_v, agg_sh, cnt_sh = rest[2 + _NBUF:6 + _NBUF]
        gsem = rest[6 + _NBUF:6 + 2 * _NBUF]
        ssem = rest[6 + 2 * _NBUF:6 + 3 * _NBUF]
        csem = rest[6 + 3 * _NBUF:6 + 4 * _NBUF]

        cid = lax.axis_index("c")
        sid = lax.axis_index("s")
        wid = sid * _NC + cid

        sbase = pl.multiple_of(wid * _EPW, 8)
        idx_cp = [pltpu.async_copy(src_hbm.at[pl.ds(sbase, _EPW)], src_v,
                                   gsem[0]),
                  pltpu.async_copy(dst_hbm.at[wid], dst_v, gsem[1])]

        z16 = _zeros16()

        def fill_zrow(i, _):
            r = i // 8
            col = (i % 8) * 16
            rows[0][r, pl.ds(col, 16)] = z16
            return 0

        lax.fori_loop(0, _CHUNK * 8, fill_zrow, 0)

        if with_cnt:
            o16 = jnp.ones((16,), jnp.float32)

            def fill_ones(i, _):
                ones_v[pl.ds(i * 16, 16)] = o16
                return 0

            lax.fori_loop(0, _CHUNK // 16, fill_ones, 0)

            def fill_zcnt(i, _):
                zcnt_v[pl.ds(i * 16, 16)] = z16
                return 0

            lax.fori_loop(0, _CNTC // 16, fill_zcnt, 0)

        _TAIL = _RPT - (_RPT // _CHUNK) * _CHUNK

        @pl.when(sid < _N // _RPT)
        def _():
            zcp = []
            for k in range(_RPT // _CHUNK):
                zcp.append(pltpu.async_copy(
                    rows[0], agg_sh.at[pl.ds(sid * _RPT + k * _CHUNK, _CHUNK)],
                    ssem[0]))
            zcp.append(pltpu.async_copy(
                rows[0].at[pl.ds(0, _TAIL)],
                agg_sh.at[pl.ds(sid * _RPT + (_RPT // _CHUNK) * _CHUNK,
                                _TAIL)], ssem[0]))
            if with_cnt:
                zcp.append(pltpu.async_copy(
                    zcnt_v, cnt_sh.at[pl.ds(sid * _CNTC, _CNTC)], ssem[0]))
            for cp in zcp:
                cp.wait()
        for cp in idx_cp:
            cp.wait()
        plsc.subcore_barrier()

        def start_gather(c, b):
            off = pl.multiple_of(c * _CHUNK, 8)
            pltpu.async_copy(x_hbm.at[src_v.at[pl.ds(off, _CHUNK)]], rows[b],
                             gsem[b])

        def wait_gather(b):
            pltpu.make_async_copy(x_hbm.at[src_v.at[pl.ds(0, _CHUNK)]],
                                  rows[b], gsem[b]).wait()

        def start_scatter(c, b):
            pltpu.async_copy(rows[b], agg_sh.at[dst_v.at[c]], ssem[b],
                             add=True)
            if with_cnt:
                pltpu.async_copy(ones_v, cnt_sh.at[dst_v.at[c]], csem[b],
                                 add=True)

        def wait_scatter(b):
            pltpu.make_async_copy(rows[b], agg_sh.at[dst_v.at[0]],
                                  ssem[b]).wait()
            if with_cnt:
                pltpu.make_async_copy(ones_v, cnt_sh.at[dst_v.at[0]],
                                      csem[b]).wait()

        for b in range(_NBUF):
            start_gather(b, b)
            if b >= 1:
                wait_gather(b - 1)
                start_scatter(b - 1, b - 1)

        def group(g, _):
            for b in range(_NBUF):
                c = g * _NBUF + b
                wait_scatter(b)
                start_gather(c, b)
                bprev = (b - 1) % _NBUF
                wait_gather(bprev)
                start_scatter(c - 1, bprev)
            return 0

        lax.fori_loop(1, _NCHUNK // _NBUF, group, 0)

        for c in range((_NCHUNK // _NBUF) * _NBUF, _NCHUNK):
            b = c % _NBUF
            wait_scatter(b)
            start_gather(c, b)
            wait_gather((b - 1) % _NBUF)
            start_scatter(c - 1, (b - 1) % _NBUF)
        blast = (_NCHUNK - 1) % _NBUF
        wait_gather(blast)
        start_scatter(_NCHUNK - 1, blast)
        for b in range(_NBUF):
            wait_scatter(b)

        plsc.subcore_barrier()

        @pl.when(sid < _N // _RPT)
        def _():
            ocp = []
            for k in range(_RPT // _ZROWS):
                rs = sid * _RPT + k * _ZROWS
                ocp.append(pltpu.async_copy(agg_sh.at[pl.ds(rs, _ZROWS)],
                                            agg_out.at[cid, pl.ds(rs, _ZROWS)],
                                            ssem[0]))
            if with_cnt:
                pltpu.sync_copy(cnt_sh.at[pl.ds(sid * _CNTC, _CNTC)], zcnt_v)
                ocp.append(pltpu.async_copy(
                    zcnt_v, cnt_out.at[pl.ds(cid * _N + sid * _CNTC, _CNTC)],
                    ssem[0]))
            for cp in ocp:
                cp.wait()

    return pl.kernel(body, out_type=out_type, mesh=mesh, scratch_types=scratch,
                     name="sc_sage_agg_cnt" if with_cnt else "sc_sage_agg")


_make_sc_agg = functools.lru_cache(maxsize=None)(_make_sc_agg)


_BM = 1000


def _make_tc_skip():
    def body(x_ref, wr_ref, b_ref, o_ref):
        o_ref[...] = (jnp.dot(x_ref[...], wr_ref[...],
                              preferred_element_type=jnp.float32)
                      + b_ref[...])

    return pl.pallas_call(
        body,
        grid=(_N // _BM,),
        in_specs=[
            pl.BlockSpec((_BM, _D), lambda i: (i, 0)),
            pl.BlockSpec((_D, _D), lambda i: (0, 0)),
            pl.BlockSpec((1, _D), lambda i: (0, 0)),
        ],
        out_specs=pl.BlockSpec((_BM, _D), lambda i: (i, 0)),
        out_shape=jax.ShapeDtypeStruct((_N, _D), jnp.float32),
        name="tc_sage_skip",
    )


def _make_tc_combine(relu: bool):
    def body(agg_ref, cnt_ref, xr_ref, wl_ref, o_ref):
        a = agg_ref[0] + agg_ref[1]
        c = cnt_ref[0] + cnt_ref[1]
        mean = a / jnp.maximum(c, 1.0)
        y = jnp.dot(mean, wl_ref[...],
                    preferred_element_type=jnp.float32) + xr_ref[...]
        if relu:
            y = jnp.maximum(y, 0.0)
        o_ref[...] = y

    return pl.pallas_call(
        body,
        grid=(_N // _BM,),
        in_specs=[
            pl.BlockSpec((_NC, _BM, _D), lambda i: (0, i, 0)),
            pl.BlockSpec((_NC, _BM, 1), lambda i: (0, i, 0)),
            pl.BlockSpec((_BM, _D), lambda i: (i, 0)),
            pl.BlockSpec((_D, _D), lambda i: (0, 0)),
        ],
        out_specs=pl.BlockSpec((_BM, _D), lambda i: (i, 0)),
        out_shape=jax.ShapeDtypeStruct((_N, _D), jnp.float32),
        name="tc_sage_combine_relu" if relu else "tc_sage_combine",
    )


_tc_skip = _make_tc_skip()
_tc_combine_relu = _make_tc_combine(True)
_tc_combine = _make_tc_combine(False)


@jax.jit
def kernel(x, edge_index, W_l1, W_r1, b1, W_l2, W_r2, b2):
    src = edge_index[0].astype(jnp.int32)
    dst = edge_index[1].astype(jnp.int32).reshape(_NW, _NCHUNK, _CHUNK)

    xr1 = _tc_skip(x, W_r1, b1.reshape(1, _D))
    agg1, cnt = _make_sc_agg(True)(src, dst, x)
    cnt3 = cnt.reshape(_NC, _N, 1)
    h = _tc_combine_relu(agg1, cnt3, xr1, W_l1)
    xr2 = _tc_skip(h, W_r2, b2.reshape(1, _D))
    (agg2,) = _make_sc_agg(False)(src, dst, h)
    out = _tc_combine(agg2, cnt3, xr2, W_l2)
    return out
